# baseline (speedup 1.0000x reference)
import jax
import jax.numpy as jnp
from jax import lax
from jax.experimental import pallas as pl
from jax.experimental.pallas import tpu as pltpu
from jax.experimental.pallas import tpu_sc as plsc

VOCAB = 1000000
MAX_LEN = 200
EMB = 64
BATCH = 4096

NC = 2
NS = 16
NW = NC * NS
BG = BATCH // NW
LANES = 16
NK = BG // LANES
EG = EMB // 8
SG = MAX_LEN // 8


def _body(xn_hbm, tab_hbm, pos_hbm, out_hbm, idx_v, sh_v, pos_v, gbuf, tbuf,
          g0, g1, o0, o1):
    c = lax.axis_index("c")
    s_ax = lax.axis_index("s")
    w = s_ax * NC + c

    pltpu.sync_copy(xn_hbm.at[:, w], idx_v)
    pltpu.sync_copy(pos_hbm, pos_v)

    gsems = (g0, g1)
    osems = (o0, o1)

    def ids_slice(s, k):
        return idx_v[s // 8, s % 8, pl.ds(k * LANES, LANES)]

    def fill_shift(s, b):
        for k in range(NK):
            sh_v[b, pl.ds(k * LANES, LANES)] = lax.shift_right_logical(
                ids_slice(s, k), 1
            )

    def start_gather(b):
        pltpu.async_copy(tab_hbm.at[sh_v.at[b]], gbuf.at[b], gsems[b])

    def wait_gather(b):
        pltpu.make_async_copy(tab_hbm.at[sh_v.at[b]], gbuf.at[b], gsems[b]).wait()

    def start_out(s, b):
        for eg in range(EG):
            pltpu.async_copy(
                tbuf.at[b, pl.ds(eg * 8, 8)], out_hbm.at[s, eg, w], osems[b]
            )

    def wait_out(b):
        for eg in range(EG):
            pltpu.make_async_copy(
                tbuf.at[b, pl.ds(eg * 8, 8)], out_hbm.at[0, eg, w], osems[b]
            ).wait()

    rowvecs = [lax.iota(jnp.int32, LANES) + jnp.int32(LANES * k) for k in range(NK)]

    def compute(s, b):
        colbases = [
            lax.shift_left(ids_slice(s, k) & 1, jnp.int32(6)) for k in range(NK)
        ]

        sv = lax.broadcast(s, (LANES,))

        def e_step(e, _):
            ev = lax.broadcast(e, (LANES,))
            pb = plsc.load_gather(pos_v, [sv, ev])
            for k in range(NK):
                val = plsc.load_gather(gbuf.at[b], [rowvecs[k], colbases[k] + ev])
                tbuf[b, e, pl.ds(k * LANES, LANES)] = val + pb
            return 0

        lax.fori_loop(0, EMB, e_step, 0, unroll=4)

    for b in range(2):
        fill_shift(b, b)
        start_gather(b)

    def step(i, _):
        for b in range(2):
            s = 2 * i + b
            wait_gather(b)

            @pl.when(i > 0)
            def _():
                wait_out(b)

            compute(s, b)
            start_out(s, b)

            @pl.when(i < MAX_LEN // 2 - 1)
            def _():
                fill_shift(s + 2, b)
                start_gather(b)

        return 0

    lax.fori_loop(0, MAX_LEN // 2, step, 0)
    wait_out(0)
    wait_out(1)


_mesh = plsc.VectorSubcoreMesh(core_axis_name="c", subcore_axis_name="s")

_emb = pl.kernel(
    _body,
    out_type=jax.ShapeDtypeStruct((MAX_LEN, EG, NW, 8, BG), jnp.float32),
    mesh=_mesh,
    compiler_params=pltpu.CompilerParams(
        use_tc_tiling_on_sc=False, needs_layout_passes=False
    ),
    scratch_types=[
        pltpu.VMEM((SG, 8, BG), jnp.int32),
        pltpu.VMEM((2, BG), jnp.int32),
        pltpu.VMEM((MAX_LEN, EMB), jnp.float32),
        pltpu.VMEM((2, BG, 128), jnp.float32),
        pltpu.VMEM((2, EMB, BG), jnp.float32),
        pltpu.SemaphoreType.DMA,
        pltpu.SemaphoreType.DMA,
        pltpu.SemaphoreType.DMA,
        pltpu.SemaphoreType.DMA,
    ],
)


@jax.jit
def kernel(x, token_table, pos_table):
    xn = (
        x.astype(jnp.int32)
        .reshape(NW, BG, SG, 8)
        .transpose(2, 0, 3, 1)
    )
    tab2 = token_table.reshape(VOCAB // 2, 128)
    out5 = _emb(xn, tab2, pos_table)
    return out5.transpose(2, 4, 0, 1, 3).reshape(BATCH, MAX_LEN, EMB)


# speedup vs baseline: 1.7663x; 1.7663x over previous
import jax
import jax.numpy as jnp
from jax import lax
from jax.experimental import pallas as pl
from jax.experimental.pallas import tpu as pltpu
from jax.experimental.pallas import tpu_sc as plsc

VOCAB = 1000000
MAX_LEN = 200
EMB = 64
BATCH = 4096

NC = 2
NS = 16
NW = NC * NS
BG = BATCH // NW
LANES = 16
NJ = EMB // LANES
EG = EMB // 8
SG = MAX_LEN // 8
TW = 133


def _body(xn_hbm, tab_hbm, pos_hbm, out_hbm, idx_v, pos_v, gbuf, tbuf,
          g0, g1, o0, o1):
    c = lax.axis_index("c")
    s_ax = lax.axis_index("s")
    w = s_ax * NC + c

    pltpu.sync_copy(xn_hbm.at[:, w], idx_v)
    pltpu.sync_copy(pos_hbm, pos_v)

    gsems = (g0, g1)
    osems = (o0, o1)

    def ids_row(s):
        return idx_v.at[s // 8, s % 8]

    def start_gather(s, b):
        pltpu.async_copy(tab_hbm.at[ids_row(s)], gbuf.at[b], gsems[b])

    def wait_gather(b):
        pltpu.make_async_copy(tab_hbm.at[ids_row(0)], gbuf.at[b], gsems[b]).wait()

    def start_out(s, b):
        for eg in range(EG):
            pltpu.async_copy(
                tbuf.at[b, pl.ds(eg * 8, 8), pl.ds(0, BG)],
                out_hbm.at[s, eg, w],
                osems[b],
            )

    def wait_out(b):
        for eg in range(EG):
            pltpu.make_async_copy(
                tbuf.at[b, pl.ds(eg * 8, 8), pl.ds(0, BG)],
                out_hbm.at[0, eg, w],
                osems[b],
            ).wait()

    rvecs = [lax.iota(jnp.int32, LANES) + jnp.int32(LANES * j2) for j2 in range(NJ)]

    def compute(s, b):
        pvecs = [pos_v[s, pl.ds(LANES * j2, LANES)] for j2 in range(NJ)]

        def tok_block(i, _):
            for jj in range(8):
                j = i * 8 + jj
                jv = lax.broadcast(j, (LANES,))
                for j2 in range(NJ):
                    val = gbuf[b, j, pl.ds(LANES * j2, LANES)] + pvecs[j2]
                    plsc.store_scatter(tbuf.at[b], [rvecs[j2], jv], val)
            return 0

        lax.fori_loop(0, BG // 8, tok_block, 0)

    start_gather(0, 0)
    start_gather(1, 1)

    def step(i, _):
        for b in range(2):
            s = 2 * i + b
            wait_gather(b)

            @pl.when(i > 0)
            def _():
                wait_out(b)

            compute(s, b)
            start_out(s, b)

            @pl.when(i < MAX_LEN // 2 - 1)
            def _():
                start_gather(s + 2, b)

        return 0

    lax.fori_loop(0, MAX_LEN // 2, step, 0)
    wait_out(0)
    wait_out(1)


_mesh = plsc.VectorSubcoreMesh(core_axis_name="c", subcore_axis_name="s")

_emb = pl.kernel(
    _body,
    out_type=jax.ShapeDtypeStruct((MAX_LEN, EG, NW, 8, BG), jnp.float32),
    mesh=_mesh,
    compiler_params=pltpu.CompilerParams(
        use_tc_tiling_on_sc=False, needs_layout_passes=False
    ),
    scratch_types=[
        pltpu.VMEM((SG, 8, BG), jnp.int32),
        pltpu.VMEM((MAX_LEN, EMB), jnp.float32),
        pltpu.VMEM((2, BG, EMB), jnp.float32),
        pltpu.VMEM((2, EMB, TW), jnp.float32),
        pltpu.SemaphoreType.DMA,
        pltpu.SemaphoreType.DMA,
        pltpu.SemaphoreType.DMA,
        pltpu.SemaphoreType.DMA,
    ],
)


@jax.jit
def kernel(x, token_table, pos_table):
    xn = (
        x.astype(jnp.int32)
        .reshape(NW, BG, SG, 8)
        .transpose(2, 0, 3, 1)
    )
    out5 = _emb(xn, token_table, pos_table)
    return out5.transpose(2, 4, 0, 1, 3).reshape(BATCH, MAX_LEN, EMB)


# speedup vs baseline: 1.9716x; 1.1162x over previous
import functools

import jax
import jax.numpy as jnp
from jax import lax
from jax.experimental import pallas as pl
from jax.experimental.pallas import tpu as pltpu
from jax.experimental.pallas import tpu_sc as plsc

VOCAB = 1000000
MAX_LEN = 200
EMB = 64
BATCH = 4096

HV = VOCAB // 2
NC = 2
NS = 16
NW = NC * NS
BG = BATCH // NW
LANES = 16
NJ = EMB // LANES
EG = EMB // 8
SG = MAX_LEN // 8
TW = 133

STRIPE = 4096
HSTRIPE = STRIPE // 2
NBLK = (VOCAB + STRIPE - 1) // STRIPE
TROWS = NBLK * HSTRIPE


def _tc_body(a_ref, out_ref):
    x = a_ref[...]
    out_ref[...] = jnp.concatenate([x[:, :HSTRIPE], x[:, HSTRIPE:]], axis=0).T


_transpose = pl.pallas_call(
    _tc_body,
    grid=(NBLK,),
    in_specs=[pl.BlockSpec((EMB, STRIPE), lambda i: (0, i))],
    out_specs=pl.BlockSpec((HSTRIPE, 2 * EMB), lambda i: (i, 0)),
    out_shape=jax.ShapeDtypeStruct((TROWS, 2 * EMB), jnp.float32),
)


def _body(xn_hbm, tab_hbm, pos_hbm, out_hbm, idx_v, sh_v, cb_v, pos_v, gbuf,
          tbuf, g0, g1, o0, o1):
    c = lax.axis_index("c")
    s_ax = lax.axis_index("s")
    w = s_ax * NC + c

    pltpu.sync_copy(xn_hbm.at[:, w], idx_v)
    pltpu.sync_copy(pos_hbm, pos_v)

    gsems = (g0, g1)
    osems = (o0, o1)

    def ids_slice(s, k):
        return idx_v[s // 8, s % 8, pl.ds(k * LANES, LANES)]

    def fill_shift(s, b):
        for k in range(BG // LANES):
            t = ids_slice(s, k)
            sh_v[b, pl.ds(k * LANES, LANES)] = (
                lax.shift_left(lax.shift_right_logical(t, 12), 11)
                + (t & jnp.int32(HSTRIPE - 1))
            )

    def start_gather(b):
        pltpu.async_copy(tab_hbm.at[sh_v.at[b]], gbuf.at[b], gsems[b])

    def wait_gather(b):
        pltpu.make_async_copy(tab_hbm.at[sh_v.at[0]], gbuf.at[b], gsems[b]).wait()

    def start_out(s, b):
        for eg in range(EG):
            pltpu.async_copy(
                tbuf.at[b, pl.ds(eg * 8, 8), pl.ds(0, BG)],
                out_hbm.at[s, eg, w],
                osems[b],
            )

    def wait_out(b):
        for eg in range(EG):
            pltpu.make_async_copy(
                tbuf.at[b, pl.ds(eg * 8, 8), pl.ds(0, BG)],
                out_hbm.at[0, eg, w],
                osems[b],
            ).wait()

    iot = lax.iota(jnp.int32, LANES)
    rvecs = [iot + jnp.int32(LANES * j2) for j2 in range(NJ)]
    cvecs = [iot + jnp.int32(LANES * j2) for j2 in range(NJ)]

    def compute(s, b):
        for k in range(BG // LANES):
            t = ids_slice(s, k)
            cb_v[pl.ds(k * LANES, LANES)] = lax.shift_left(
                lax.shift_right_logical(t, 11) & 1, jnp.int32(6)
            )
        pvecs = [pos_v[s, pl.ds(LANES * j2, LANES)] for j2 in range(NJ)]

        def tok_block(i, _):
            for jj in range(8):
                j = i * 8 + jj
                jv = lax.broadcast(j, (LANES,))
                cbs = plsc.load_gather(cb_v, [jv])
                for j2 in range(NJ):
                    val = plsc.load_gather(gbuf.at[b], [jv, cbs + cvecs[j2]])
                    plsc.store_scatter(
                        tbuf.at[b], [rvecs[j2], jv], val + pvecs[j2]
                    )
            return 0

        lax.fori_loop(0, BG // 8, tok_block, 0)

    for b in range(2):
        fill_shift(b, b)
        start_gather(b)

    def step(i, _):
        for b in range(2):
            s = 2 * i + b
            wait_gather(b)

            @pl.when(i > 0)
            def _():
                wait_out(b)

            compute(s, b)
            start_out(s, b)

            @pl.when(i < MAX_LEN // 2 - 1)
            def _():
                fill_shift(s + 2, b)
                start_gather(b)

        return 0

    lax.fori_loop(0, MAX_LEN // 2, step, 0)
    wait_out(0)
    wait_out(1)


_mesh = plsc.VectorSubcoreMesh(core_axis_name="c", subcore_axis_name="s")

_emb = pl.kernel(
    _body,
    out_type=jax.ShapeDtypeStruct((MAX_LEN, EG, NW, 8, BG), jnp.float32),
    mesh=_mesh,
    compiler_params=pltpu.CompilerParams(
        use_tc_tiling_on_sc=False, needs_layout_passes=False
    ),
    scratch_types=[
        pltpu.VMEM((SG, 8, BG), jnp.int32),
        pltpu.VMEM((2, BG), jnp.int32),
        pltpu.VMEM((BG,), jnp.int32),
        pltpu.VMEM((MAX_LEN, EMB), jnp.float32),
        pltpu.VMEM((2, BG, 2 * EMB), jnp.float32),
        pltpu.VMEM((2, EMB, TW), jnp.float32),
        pltpu.SemaphoreType.DMA,
        pltpu.SemaphoreType.DMA,
        pltpu.SemaphoreType.DMA,
        pltpu.SemaphoreType.DMA,
    ],
)


@jax.jit
def kernel(x, token_table, pos_table):
    xn = (
        x.astype(jnp.int32)
        .reshape(NW, BG, SG, 8)
        .transpose(2, 0, 3, 1)
    )
    tt = jnp.swapaxes(token_table, 0, 1)
    tab2 = _transpose(tt)
    out5 = _emb(xn, tab2, pos_table)
    return out5.transpose(2, 4, 0, 1, 3).reshape(BATCH, MAX_LEN, EMB)


# speedup vs baseline: 2.4192x; 1.2271x over previous
import functools

import jax
import jax.numpy as jnp
from jax import lax
from jax.experimental import pallas as pl
from jax.experimental.pallas import tpu as pltpu
from jax.experimental.pallas import tpu_sc as plsc

VOCAB = 1000000
MAX_LEN = 200
EMB = 64
BATCH = 4096

HV = VOCAB // 2
NC = 2
NS = 16
NW = NC * NS
BG = BATCH // NW
LANES = 16
NJ = EMB // LANES
EG = EMB // 8
SG = MAX_LEN // 8
TW = 133

STRIPE = 4096
HSTRIPE = STRIPE // 2
NBLK = (VOCAB + STRIPE - 1) // STRIPE
TROWS = NBLK * HSTRIPE


def _tc_body(a_ref, out_ref):
    x = a_ref[...]
    out_ref[...] = jnp.concatenate([x[:, :HSTRIPE], x[:, HSTRIPE:]], axis=0).T


_transpose = pl.pallas_call(
    _tc_body,
    grid=(NBLK,),
    in_specs=[pl.BlockSpec((EMB, STRIPE), lambda i: (0, i))],
    out_specs=pl.BlockSpec((HSTRIPE, 2 * EMB), lambda i: (i, 0)),
    out_shape=jax.ShapeDtypeStruct((TROWS, 2 * EMB), jnp.float32),
)


def _body(xn_hbm, tab_hbm, pos_hbm, out_hbm, idx_v, sh_v, cb_v, pos_v, gbuf,
          tbuf, g0, g1, o0, o1):
    c = lax.axis_index("c")
    s_ax = lax.axis_index("s")
    w = s_ax * NC + c

    pltpu.sync_copy(xn_hbm.at[:, w], idx_v)
    pltpu.sync_copy(pos_hbm, pos_v)

    gsems = (g0, g1)
    osems = (o0, o1)

    def ids_slice(s, k):
        return idx_v[s // 8, s % 8, pl.ds(k * LANES, LANES)]

    def fill_shift(s, b):
        for k in range(BG // LANES):
            t = ids_slice(s, k)
            sh_v[b, pl.ds(k * LANES, LANES)] = (
                lax.shift_left(lax.shift_right_logical(t, 12), 11)
                + (t & jnp.int32(HSTRIPE - 1))
            )

    def start_gather(b):
        pltpu.async_copy(tab_hbm.at[sh_v.at[b]], gbuf.at[b], gsems[b])

    def wait_gather(b):
        pltpu.make_async_copy(tab_hbm.at[sh_v.at[0]], gbuf.at[b], gsems[b]).wait()

    def start_out(s, b):
        for eg in range(EG):
            pltpu.async_copy(
                tbuf.at[b, pl.ds(eg * 8, 8), pl.ds(0, BG)],
                out_hbm.at[s, eg, w],
                osems[b],
            )

    def wait_out(b):
        for eg in range(EG):
            pltpu.make_async_copy(
                tbuf.at[b, pl.ds(eg * 8, 8), pl.ds(0, BG)],
                out_hbm.at[0, eg, w],
                osems[b],
            ).wait()

    iot = lax.iota(jnp.int32, LANES)
    rvecs = [iot + jnp.int32(LANES * j2) for j2 in range(NJ)]
    cvecs = [iot + jnp.int32(LANES * j2) for j2 in range(NJ)]

    def compute(s, b):
        for k in range(BG // LANES):
            t = ids_slice(s, k)
            cb_v[pl.ds(k * LANES, LANES)] = lax.shift_left(
                lax.shift_right_logical(t, 11) & 1, jnp.int32(6)
            )
        pvecs = [pos_v[s, pl.ds(LANES * j2, LANES)] for j2 in range(NJ)]

        def tok_block(i, _):
            cb_vec = cb_v[pl.ds(i * LANES, LANES)]
            for jj in range(LANES):
                jv = lax.broadcast(i * LANES + jj, (LANES,))
                cbs = lax.broadcast(cb_vec[jj], (LANES,))
                for j2 in range(NJ):
                    val = plsc.load_gather(gbuf.at[b], [jv, cbs + cvecs[j2]])
                    plsc.store_scatter(
                        tbuf.at[b], [rvecs[j2], jv], val + pvecs[j2]
                    )
            return 0

        lax.fori_loop(0, BG // LANES, tok_block, 0)

    for b in range(2):
        fill_shift(b, b)
        start_gather(b)

    def step(i, _):
        for b in range(2):
            s = 2 * i + b
            wait_gather(b)

            @pl.when(i > 0)
            def _():
                wait_out(b)

            compute(s, b)
            start_out(s, b)

            @pl.when(i < MAX_LEN // 2 - 1)
            def _():
                fill_shift(s + 2, b)
                start_gather(b)

        return 0

    lax.fori_loop(0, MAX_LEN // 2, step, 0)
    wait_out(0)
    wait_out(1)


_mesh = plsc.VectorSubcoreMesh(core_axis_name="c", subcore_axis_name="s")

_emb = pl.kernel(
    _body,
    out_type=jax.ShapeDtypeStruct((MAX_LEN, EG, NW, 8, BG), jnp.float32),
    mesh=_mesh,
    compiler_params=pltpu.CompilerParams(
        use_tc_tiling_on_sc=False, needs_layout_passes=False
    ),
    scratch_types=[
        pltpu.VMEM((SG, 8, BG), jnp.int32),
        pltpu.VMEM((2, BG), jnp.int32),
        pltpu.VMEM((BG,), jnp.int32),
        pltpu.VMEM((MAX_LEN, EMB), jnp.float32),
        pltpu.VMEM((2, BG, 2 * EMB), jnp.float32),
        pltpu.VMEM((2, EMB, TW), jnp.float32),
        pltpu.SemaphoreType.DMA,
        pltpu.SemaphoreType.DMA,
        pltpu.SemaphoreType.DMA,
        pltpu.SemaphoreType.DMA,
    ],
)


@jax.jit
def kernel(x, token_table, pos_table):
    xn = (
        x.astype(jnp.int32)
        .reshape(NW, BG, SG, 8)
        .transpose(2, 0, 3, 1)
    )
    tt = jnp.swapaxes(token_table, 0, 1)
    tab2 = _transpose(tt)
    out5 = _emb(xn, tab2, pos_table)
    return out5.transpose(2, 4, 0, 1, 3).reshape(BATCH, MAX_LEN, EMB)
